# initial kernel scaffold (unmeasured)
import jax
import jax.numpy as jnp
from jax import lax
from jax.experimental import pallas as pl
from jax.experimental.pallas import tpu as pltpu

N_DEV = 4
NC = 1024


def kernel(x, w_mat):
    m, k_shard = x.shape
    _, n = w_mat.shape
    grid = n // NC

    def body(x_ref, w_ref, out_ref, comm_ref, send_sems, recv_sems):
        my = lax.axis_index("i")
        left = (my + N_DEV - 1) % N_DEV
        right = (my + 1) % N_DEV

        barrier_sem = pltpu.get_barrier_semaphore()
        for nbr in (left, right):
            pl.semaphore_signal(
                barrier_sem, inc=1,
                device_id=(nbr,), device_id_type=pl.DeviceIdType.MESH,
            )
        pl.semaphore_wait(barrier_sem, 2)

        partial = jnp.dot(
            x_ref[...], w_ref[...], preferred_element_type=jnp.float32
        )
        comm_ref[0] = partial.astype(jnp.bfloat16)
        acc = partial

        for h in range(N_DEV - 1):
            rdma = pltpu.make_async_remote_copy(
                src_ref=comm_ref.at[h],
                dst_ref=comm_ref.at[h + 1],
                send_sem=send_sems.at[h],
                recv_sem=recv_sems.at[h],
                device_id=(right,),
                device_id_type=pl.DeviceIdType.MESH,
            )
            rdma.start()
            rdma.wait()
            acc = acc + comm_ref[h + 1].astype(jnp.float32)

        out_ref[...] = acc

    return pl.pallas_call(
        body,
        grid=(grid,),
        out_shape=jax.ShapeDtypeStruct((m, n), jnp.float32),
        in_specs=[
            pl.BlockSpec((m, k_shard), lambda j: (0, 0)),
            pl.BlockSpec((k_shard, NC), lambda j: (0, j)),
        ],
        out_specs=pl.BlockSpec((m, NC), lambda j: (0, j)),
        scratch_shapes=[
            pltpu.VMEM((N_DEV, m, NC), jnp.bfloat16),
            pltpu.SemaphoreType.DMA((N_DEV - 1,)),
            pltpu.SemaphoreType.DMA((N_DEV - 1,)),
        ],
        compiler_params=pltpu.CompilerParams(
            collective_id=0,
            dimension_semantics=("arbitrary",),
        ),
    )(x, w_mat)


# baseline (device time: 2580054 ns/iter reference)
import jax
import jax.numpy as jnp
from jax import lax
from jax.experimental import pallas as pl
from jax.experimental.pallas import tpu as pltpu

N_DEV = 4
NC = 256


def kernel(x, w_mat):
    x = x.astype(jnp.bfloat16)
    w_mat = w_mat.astype(jnp.bfloat16)
    m, k_shard = x.shape
    _, n = w_mat.shape
    grid = n // NC

    def body(x_ref, w_ref, out_ref, comm_ref, send_sems, recv_sems):
        my = lax.axis_index("i")
        left = (my + N_DEV - 1) % N_DEV
        right = (my + 1) % N_DEV

        barrier_sem = pltpu.get_barrier_semaphore()
        for nbr in (left, right):
            pl.semaphore_signal(
                barrier_sem, inc=1,
                device_id=(nbr,), device_id_type=pl.DeviceIdType.MESH,
            )
        pl.semaphore_wait(barrier_sem, 2)

        partial = jnp.dot(
            x_ref[...], w_ref[...], preferred_element_type=jnp.float32
        )
        comm_ref[0] = partial.astype(jnp.bfloat16)
        out_ref[...] = partial

        for h in range(N_DEV - 1):
            rdma = pltpu.make_async_remote_copy(
                src_ref=comm_ref.at[h],
                dst_ref=comm_ref.at[h + 1],
                send_sem=send_sems.at[h],
                recv_sem=recv_sems.at[h],
                device_id=(right,),
                device_id_type=pl.DeviceIdType.MESH,
            )
            rdma.start()
            rdma.wait()
            out_ref[...] += comm_ref[h + 1].astype(jnp.float32)

    return pl.pallas_call(
        body,
        grid=(grid,),
        out_shape=jax.ShapeDtypeStruct((m, n), jnp.float32),
        in_specs=[
            pl.BlockSpec((m, k_shard), lambda j: (0, 0)),
            pl.BlockSpec((k_shard, NC), lambda j: (0, j)),
        ],
        out_specs=pl.BlockSpec((m, NC), lambda j: (0, j)),
        scratch_shapes=[
            pltpu.VMEM((N_DEV, m, NC), jnp.bfloat16),
            pltpu.SemaphoreType.DMA((N_DEV - 1,)),
            pltpu.SemaphoreType.DMA((N_DEV - 1,)),
        ],
        compiler_params=pltpu.CompilerParams(
            collective_id=0,
            dimension_semantics=("arbitrary",),
        ),
    )(x, w_mat)


# device time: 901676 ns/iter; 2.8614x vs baseline; 2.8614x over previous
import jax
import jax.numpy as jnp
from jax import lax
from jax.experimental import pallas as pl
from jax.experimental.pallas import tpu as pltpu

N_DEV = 4
NC = 512


def kernel(x, w_mat):
    x = x.astype(jnp.bfloat16)
    w_mat = w_mat.astype(jnp.bfloat16)
    m, k_shard = x.shape
    _, n = w_mat.shape
    grid = n // NC
    W = NC // 2
    H = m // 2
    Q = m // 4

    def body(x_ref, w_ref, out_ref, part, s1r, har, s2r, qar, q3r, h4q, h4r,
             ssem, rsem):
        my = lax.axis_index("i")

        barrier_sem = pltpu.get_barrier_semaphore()
        for nbr in (my ^ 1, 3 - my):
            pl.semaphore_signal(
                barrier_sem, inc=1,
                device_id=(nbr,), device_id_type=pl.DeviceIdType.MESH,
            )
        pl.semaphore_wait(barrier_sem, 2)

        for rb in range(4):
            rows = pl.ds(rb * Q, Q)
            t = jnp.dot(
                x_ref[rows, :], w_ref[...],
                preferred_element_type=jnp.float32,
            )
            part[0, rows] = t[:, :W].astype(jnp.bfloat16)
            part[1, rows] = t[:, W:].astype(jnp.bfloat16)

        def params(c):
            if c == 0:
                p1, p2 = my ^ 1, 3 - my
                h = (my & 1) ^ (my >> 1)
                q = my >> 1
                qp = q
            else:
                p1, p2 = 3 - my, my ^ 1
                h = my >> 1
                q = my & 1
                qp = 1 - q
            return p1, p2, h, q, qp

        def exchange(srcs, dsts, devs, sem_idx):
            rdmas = []
            for c in (0, 1):
                r = pltpu.make_async_remote_copy(
                    src_ref=srcs[c],
                    dst_ref=dsts[c],
                    send_sem=ssem.at[c, sem_idx],
                    recv_sem=rsem.at[c, sem_idx],
                    device_id=(devs[c],),
                    device_id_type=pl.DeviceIdType.MESH,
                )
                r.start()
                rdmas.append(r)
            for r in rdmas:
                r.wait()

        P = [params(0), params(1)]

        exchange(
            [part.at[c, pl.ds((1 - P[c][2]) * H, H)] for c in (0, 1)],
            [s1r.at[c] for c in (0, 1)],
            [P[c][0] for c in (0, 1)],
            0,
        )
        for c in (0, 1):
            h = P[c][2]
            har[c] = part[c, pl.ds(h * H, H)] + s1r[c]

        exchange(
            [har.at[c, pl.ds((1 - P[c][3]) * Q, Q)] for c in (0, 1)],
            [s2r.at[c] for c in (0, 1)],
            [P[c][1] for c in (0, 1)],
            1,
        )
        for c in (0, 1):
            q = P[c][3]
            qar[c] = har[c, pl.ds(q * Q, Q)] + s2r[c]

        exchange(
            [qar.at[c] for c in (0, 1)],
            [q3r.at[c] for c in (0, 1)],
            [P[c][1] for c in (0, 1)],
            2,
        )

        exchange(
            [qar.at[c] for c in (0, 1)],
            [h4q.at[c] for c in (0, 1)],
            [P[c][0] for c in (0, 1)],
            3,
        )
        exchange(
            [q3r.at[c] for c in (0, 1)],
            [h4r.at[c] for c in (0, 1)],
            [P[c][0] for c in (0, 1)],
            4,
        )

        for c in (0, 1):
            _, _, h, q, qp = P[c]
            cols = pl.ds(c * W, W)
            out_ref[pl.ds((2 * h + q) * Q, Q), cols] = qar[c].astype(
                jnp.float32)
            out_ref[pl.ds((2 * h + (1 - q)) * Q, Q), cols] = q3r[c].astype(
                jnp.float32)
            out_ref[pl.ds((2 * (1 - h) + qp) * Q, Q), cols] = h4q[c].astype(
                jnp.float32)
            out_ref[pl.ds((2 * (1 - h) + (1 - qp)) * Q, Q), cols] = h4r[
                c].astype(jnp.float32)

    return pl.pallas_call(
        body,
        grid=(grid,),
        out_shape=jax.ShapeDtypeStruct((m, n), jnp.float32),
        in_specs=[
            pl.BlockSpec((m, k_shard), lambda j: (0, 0)),
            pl.BlockSpec((k_shard, NC), lambda j: (0, j)),
        ],
        out_specs=pl.BlockSpec((m, NC), lambda j: (0, j)),
        scratch_shapes=[
            pltpu.VMEM((2, m, W), jnp.bfloat16),
            pltpu.VMEM((2, H, W), jnp.bfloat16),
            pltpu.VMEM((2, H, W), jnp.bfloat16),
            pltpu.VMEM((2, Q, W), jnp.bfloat16),
            pltpu.VMEM((2, Q, W), jnp.bfloat16),
            pltpu.VMEM((2, Q, W), jnp.bfloat16),
            pltpu.VMEM((2, Q, W), jnp.bfloat16),
            pltpu.VMEM((2, Q, W), jnp.bfloat16),
            pltpu.SemaphoreType.DMA((2, 5)),
            pltpu.SemaphoreType.DMA((2, 5)),
        ],
        compiler_params=pltpu.CompilerParams(
            collective_id=0,
            dimension_semantics=("arbitrary",),
            vmem_limit_bytes=100 * 1024 * 1024,
        ),
    )(x, w_mat)


# device time: 736974 ns/iter; 3.5009x vs baseline; 1.2235x over previous
import jax
import jax.numpy as jnp
from jax import lax
from jax.experimental import pallas as pl
from jax.experimental.pallas import tpu as pltpu

N_DEV = 4
NC = 512


def kernel(x, w_mat):
    x = x.astype(jnp.bfloat16)
    w_mat = w_mat.astype(jnp.bfloat16)
    m, k_shard = x.shape
    _, n = w_mat.shape
    G = n // NC
    W = NC // 2
    H = m // 2
    Q = m // 4

    def body(x_ref, w_ref, out_ref, part, s1r, har, s2r, qar, q3r, h4q, h4r,
             ssem, rsem):
        my = lax.axis_index("i")
        j = pl.program_id(0)
        pb = j % 2
        pbm = 1 - pb

        def params(c):
            if c == 0:
                p1, p2 = my ^ 1, 3 - my
                h = (my & 1) ^ (my >> 1)
                q = my >> 1
                qp = q
            else:
                p1, p2 = 3 - my, my ^ 1
                h = my >> 1
                q = my & 1
                qp = 1 - q
            return p1, p2, h, q, qp

        P = [params(0), params(1)]

        def copy(src, dst, c, k, dev):
            return pltpu.make_async_remote_copy(
                src_ref=src, dst_ref=dst,
                send_sem=ssem.at[c, k], recv_sem=rsem.at[c, k],
                device_id=(dev,), device_id_type=pl.DeviceIdType.MESH,
            )

        r1 = [copy(part.at[c, pl.ds((1 - P[c][2]) * H, H)],
                   s1r.at[pb, c], c, pb, P[c][0]) for c in (0, 1)]
        r2_new = [copy(har.at[c, pl.ds((1 - P[c][3]) * Q, Q)],
                       s2r.at[pb, c], c, 2 + pb, P[c][1]) for c in (0, 1)]
        r2_old = [copy(har.at[c, pl.ds((1 - P[c][3]) * Q, Q)],
                       s2r.at[pbm, c], c, 2 + pbm, P[c][1]) for c in (0, 1)]
        r3 = [copy(qar.at[c], q3r.at[c], c, 4, P[c][1]) for c in (0, 1)]
        r4a = [copy(qar.at[c], h4q.at[c], c, 5, P[c][0]) for c in (0, 1)]
        r4b = [copy(q3r.at[c], h4r.at[c], c, 6, P[c][0]) for c in (0, 1)]

        barrier_sem = pltpu.get_barrier_semaphore()

        @pl.when(j == 0)
        def _entry_barrier():
            for nbr in (my ^ 1, 3 - my):
                pl.semaphore_signal(
                    barrier_sem, inc=1,
                    device_id=(nbr,), device_id_type=pl.DeviceIdType.MESH,
                )
            pl.semaphore_wait(barrier_sem, 2)

        @pl.when(j < G)
        def _compute_and_ex1():
            for rb in range(4):
                rows = pl.ds(rb * Q, Q)
                t = jnp.dot(
                    x_ref[rows, :], w_ref[...],
                    preferred_element_type=jnp.float32,
                )
                part[0, rows] = t[:, :W].astype(jnp.bfloat16)
                part[1, rows] = t[:, W:].astype(jnp.bfloat16)
            for c in (0, 1):
                r1[c].start()

        @pl.when(j >= 1)
        def _finish_prev():
            for c in (0, 1):
                r2_old[c].wait()
            for c in (0, 1):
                q = P[c][3]
                qar[c] = har[c, pl.ds(q * Q, Q)] + s2r[pbm, c]
            for c in (0, 1):
                r3[c].start()
            for c in (0, 1):
                r3[c].wait()
            for c in (0, 1):
                r4a[c].start()
                r4b[c].start()
            for c in (0, 1):
                r4a[c].wait()
                r4b[c].wait()
            for c in (0, 1):
                _, _, h, q, qp = P[c]
                cols = pl.ds(c * W, W)
                out_ref[pl.ds((2 * h + q) * Q, Q), cols] = (
                    qar[c].astype(jnp.float32))
                out_ref[pl.ds((2 * h + (1 - q)) * Q, Q), cols] = (
                    q3r[c].astype(jnp.float32))
                out_ref[pl.ds((2 * (1 - h) + qp) * Q, Q), cols] = (
                    h4q[c].astype(jnp.float32))
                out_ref[pl.ds((2 * (1 - h) + (1 - qp)) * Q, Q), cols] = (
                    h4r[c].astype(jnp.float32))

        @pl.when(j < G)
        def _ex1_wait_ex2_start():
            for c in (0, 1):
                r1[c].wait()
            for c in (0, 1):
                h = P[c][2]
                har[c] = part[c, pl.ds(h * H, H)] + s1r[pb, c]
            for c in (0, 1):
                r2_new[c].start()

    return pl.pallas_call(
        body,
        grid=(G + 1,),
        out_shape=jax.ShapeDtypeStruct((m, n), jnp.float32),
        in_specs=[
            pl.BlockSpec((m, k_shard), lambda j: (0, 0)),
            pl.BlockSpec((k_shard, NC), lambda j: (0, jnp.minimum(j, G - 1))),
        ],
        out_specs=pl.BlockSpec((m, NC), lambda j: (0, jnp.maximum(j - 1, 0))),
        scratch_shapes=[
            pltpu.VMEM((2, m, W), jnp.bfloat16),
            pltpu.VMEM((2, 2, H, W), jnp.bfloat16),
            pltpu.VMEM((2, H, W), jnp.bfloat16),
            pltpu.VMEM((2, 2, Q, W), jnp.bfloat16),
            pltpu.VMEM((2, Q, W), jnp.bfloat16),
            pltpu.VMEM((2, Q, W), jnp.bfloat16),
            pltpu.VMEM((2, Q, W), jnp.bfloat16),
            pltpu.VMEM((2, Q, W), jnp.bfloat16),
            pltpu.SemaphoreType.DMA((2, 7)),
            pltpu.SemaphoreType.DMA((2, 7)),
        ],
        compiler_params=pltpu.CompilerParams(
            collective_id=0,
            dimension_semantics=("arbitrary",),
            vmem_limit_bytes=100 * 1024 * 1024,
        ),
    )(x, w_mat)


# device time: 662703 ns/iter; 3.8932x vs baseline; 1.1121x over previous
import jax
import jax.numpy as jnp
from jax import lax
from jax.experimental import pallas as pl
from jax.experimental.pallas import tpu as pltpu

N_DEV = 4
NC = 1024


def kernel(x, w_mat):
    x = x.astype(jnp.bfloat16)
    w_mat = w_mat.astype(jnp.bfloat16)
    m, k_shard = x.shape
    _, n = w_mat.shape
    G = n // NC
    W = NC // 2
    H = m // 2
    Q = m // 4

    def body(x_hbm, w_ref, out_ref, x_vmem, part, s1r, har, s2r, qar,
             ssem, rsem, xsem):
        my = lax.axis_index("i")
        j = pl.program_id(0)
        pb = j % 2
        pbm = 1 - pb

        def params(c):
            if c == 0:
                p1, p2 = my ^ 1, 3 - my
                h = (my & 1) ^ (my >> 1)
                q = my >> 1
                qp = q
            else:
                p1, p2 = 3 - my, my ^ 1
                h = my >> 1
                q = my & 1
                qp = 1 - q
            return p1, p2, h, q, qp

        P = [params(0), params(1)]

        def copy(src, dst, c, k, dev):
            return pltpu.make_async_remote_copy(
                src_ref=src, dst_ref=dst,
                send_sem=ssem.at[c, k], recv_sem=rsem.at[c, k],
                device_id=(dev,), device_id_type=pl.DeviceIdType.MESH,
            )

        def myq_rows(c):
            _, _, h, q, _ = P[c]
            return pl.ds((2 * h + q) * Q, Q)

        def otherq_rows(c):
            _, _, h, q, _ = P[c]
            return pl.ds((2 * h + (1 - q)) * Q, Q)

        r1 = [copy(part.at[c, pl.ds((1 - P[c][2]) * H, H)],
                   s1r.at[pb, c], c, pb, P[c][0]) for c in (0, 1)]
        r2_new = [copy(har.at[c, pl.ds((1 - P[c][3]) * Q, Q)],
                       s2r.at[pb, c], c, 2 + pb, P[c][1]) for c in (0, 1)]
        r2_old = [copy(har.at[c, pl.ds((1 - P[c][3]) * Q, Q)],
                       s2r.at[pbm, c], c, 2 + pbm, P[c][1]) for c in (0, 1)]
        cols = [pl.ds(c * W, W) for c in (0, 1)]
        r3 = [copy(qar.at[c], out_ref.at[myq_rows(c), cols[c]],
                   c, 4, P[c][1]) for c in (0, 1)]
        r4a = [copy(qar.at[c], out_ref.at[myq_rows(c), cols[c]],
                    c, 5, P[c][0]) for c in (0, 1)]
        r4b = [copy(out_ref.at[otherq_rows(c), cols[c]],
                    out_ref.at[otherq_rows(c), cols[c]],
                    c, 6, P[c][0]) for c in (0, 1)]

        barrier_sem = pltpu.get_barrier_semaphore()

        @pl.when(j == 0)
        def _entry():
            xcp = pltpu.make_async_copy(x_hbm, x_vmem, xsem)
            xcp.start()
            for nbr in (my ^ 1, 3 - my):
                pl.semaphore_signal(
                    barrier_sem, inc=1,
                    device_id=(nbr,), device_id_type=pl.DeviceIdType.MESH,
                )
            pl.semaphore_wait(barrier_sem, 2)
            xcp.wait()

        @pl.when(j < G)
        def _compute_and_ex1():
            for rb in range(8):
                rows = pl.ds(rb * (m // 8), m // 8)
                t = jnp.dot(
                    x_vmem[rows, :], w_ref[...],
                    preferred_element_type=jnp.float32,
                )
                part[0, rows] = t[:, :W].astype(jnp.bfloat16)
                part[1, rows] = t[:, W:].astype(jnp.bfloat16)
            for c in (0, 1):
                r1[c].start()

        @pl.when(j >= 1)
        def _finish_prev():
            for c in (0, 1):
                r2_old[c].wait()
            for c in (0, 1):
                q = P[c][3]
                qar[c] = har[c, pl.ds(q * Q, Q)] + s2r[pbm, c]
            for c in (0, 1):
                r3[c].start()
                r4a[c].start()
            for c in (0, 1):
                out_ref[myq_rows(c), cols[c]] = qar[c]
            for c in (0, 1):
                r3[c].wait()
            for c in (0, 1):
                r4b[c].start()
            for c in (0, 1):
                r4a[c].wait()
                r4b[c].wait()

        @pl.when(j < G)
        def _ex1_wait_ex2_start():
            for c in (0, 1):
                r1[c].wait()
            for c in (0, 1):
                h = P[c][2]
                har[c] = part[c, pl.ds(h * H, H)] + s1r[pb, c]
            for c in (0, 1):
                r2_new[c].start()

    return pl.pallas_call(
        body,
        grid=(G + 1,),
        out_shape=jax.ShapeDtypeStruct((m, n), jnp.bfloat16),
        in_specs=[
            pl.BlockSpec(memory_space=pl.ANY),
            pl.BlockSpec((k_shard, NC), lambda j: (0, jnp.minimum(j, G - 1))),
        ],
        out_specs=pl.BlockSpec((m, NC), lambda j: (0, jnp.maximum(j - 1, 0))),
        scratch_shapes=[
            pltpu.VMEM((m, k_shard), jnp.bfloat16),
            pltpu.VMEM((2, m, W), jnp.bfloat16),
            pltpu.VMEM((2, 2, H, W), jnp.bfloat16),
            pltpu.VMEM((2, H, W), jnp.bfloat16),
            pltpu.VMEM((2, 2, Q, W), jnp.bfloat16),
            pltpu.VMEM((2, Q, W), jnp.bfloat16),
            pltpu.SemaphoreType.DMA((2, 7)),
            pltpu.SemaphoreType.DMA((2, 7)),
            pltpu.SemaphoreType.DMA,
        ],
        compiler_params=pltpu.CompilerParams(
            collective_id=0,
            dimension_semantics=("arbitrary",),
            vmem_limit_bytes=100 * 1024 * 1024,
        ),
    )(x, w_mat)
